# baseline (speedup 1.0000x reference)
import functools

import jax
import jax.numpy as jnp
from jax import lax
from jax.experimental import pallas as pl
from jax.experimental.pallas import tpu as pltpu
from jax.experimental.pallas import tpu_sc as plsc

_EPS = 1e-5
_BOUNDARY = 1.0 - _EPS
_VOCAB = 1000000
_DIM = 32
_BATCH = 16384
_SAMPLES = 50

_NC, _NS = 2, 16
_NW = _NC * _NS
_BPW = _BATCH // _NW
_CB = 16
_NCH = _BPW // _CB
_HL = _DIM // 2


def _sc_body(table_hbm, idx_hbm, n2_hbm, dot_hbm,
             idx_v0, idx_v1, rows_v0, rows_v1, n2_v, dot_v, sem):
    wid = lax.axis_index("s") * _NC + lax.axis_index("c")
    base = wid * _BPW
    lanes = lax.iota(jnp.int32, 16)
    zero16 = jnp.zeros((16,), jnp.float32)
    idx_bufs = (idx_v0, idx_v1)
    row_bufs = (rows_v0, rows_v1)

    def fire(cc, idx_v, rows_v):
        b0 = base + cc * _CB
        pltpu.sync_copy(idx_hbm.at[pl.ds(b0, _CB)], idx_v)
        for b in range(_CB):
            pltpu.async_copy(
                table_hbm.at[idx_v.at[b]],
                rows_v.at[pl.ds(b * _SAMPLES, _SAMPLES)], sem)

    def compute(cc, rows_v):
        b0 = base + cc * _CB

        @plsc.parallel_loop(0, _CB, 1)
        def _bloop(b, rows_v=rows_v):
            rb = b * _SAMPLES
            bvec = lanes * 0 + b
            rbvec = lanes * 0 + rb
            svecs = [jnp.minimum(lanes + 16 * g, _SAMPLES - 1)
                     for g in range(4)]
            rvecs = [sv + rb for sv in svecs]

            @plsc.parallel_loop(0, _DIM, 1, unroll=8, carry=(zero16,) * 8)
            def acc(d, acc, rvecs=rvecs, rbvec=rbvec):
                dvec = lanes * 0 + d
                u_d = plsc.load_gather(rows_v, [rbvec, dvec])
                outs = []
                for g in range(4):
                    v = plsc.load_gather(rows_v, [rvecs[g], dvec])
                    outs.append(acc[2 * g] + v * v)
                    outs.append(acc[2 * g + 1] + u_d * v)
                return tuple(outs)

            for g in range(4):
                plsc.store_scatter(n2_v, [bvec, svecs[g]], acc[2 * g])
                plsc.store_scatter(
                    dot_v, [bvec, svecs[g] - 1], acc[2 * g + 1],
                    mask=svecs[g] >= 1)
        pltpu.sync_copy(n2_v, n2_hbm.at[pl.ds(b0, _CB)])
        pltpu.sync_copy(dot_v, dot_hbm.at[pl.ds(b0, _CB)])

    def drain(rows_v):
        pltpu.make_async_copy(table_hbm.at[pl.ds(0, _CB * _SAMPLES)],
                              rows_v, sem).wait()

    fire(0, idx_bufs[0], row_bufs[0])

    def cbody(co, carry):
        for h in (0, 1):
            cc = 2 * co + h

            @pl.when(cc + 1 < _NCH)
            def _fire_next(cc=cc, h=h):
                fire(cc + 1, idx_bufs[h ^ 1], row_bufs[h ^ 1])

            drain(row_bufs[h])
            compute(cc, row_bufs[h])
        return carry

    lax.fori_loop(0, _NCH // 2, cbody, 0)


@functools.cache
def _sc_gather_reduce():
    return pl.kernel(
        _sc_body,
        out_type=(
            jax.ShapeDtypeStruct((_BATCH, _SAMPLES), jnp.float32),
            jax.ShapeDtypeStruct((_BATCH, _SAMPLES - 1), jnp.float32),
        ),
        mesh=plsc.VectorSubcoreMesh(
            core_axis_name="c", subcore_axis_name="s",
            num_cores=_NC, num_subcores=_NS,
        ),
        scratch_types=[
            pltpu.VMEM((_CB, _SAMPLES), jnp.int32),
            pltpu.VMEM((_CB, _SAMPLES), jnp.int32),
            pltpu.VMEM((_CB * _SAMPLES, _DIM), jnp.float32),
            pltpu.VMEM((_CB * _SAMPLES, _DIM), jnp.float32),
            pltpu.VMEM((_CB, _SAMPLES), jnp.float32),
            pltpu.VMEM((_CB, _SAMPLES - 1), jnp.float32),
            pltpu.SemaphoreType.DMA,
        ],
        compiler_params=pltpu.CompilerParams(
            use_tc_tiling_on_sc=False, needs_layout_passes=False),
    )


_BB = 2048


def _fin_body(n2_ref, dot_ref, out_ref):
    n2 = n2_ref[...]
    dt = dot_ref[...]
    n = jnp.sqrt(n2)
    scale = jnp.where(n > 1.0, 1.0 / (n + 1e-7), 1.0)
    sq_raw = n2 * scale * scale
    sq = jnp.clip(sq_raw, 0.0, _BOUNDARY)
    u2 = sq_raw[:, :1]
    v2 = sq_raw[:, 1:]
    su = scale[:, :1]
    sv = scale[:, 1:]
    sqdist = u2 + v2 - 2.0 * (su * sv) * dt
    squ = sq[:, :1]
    sqv = sq[:, 1:]
    x = sqdist / ((1.0 - squ) * (1.0 - sqv)) * 2.0 + 1.0
    z = jnp.sqrt(jnp.maximum(x * x - 1.0, 1e-12))
    out_ref[...] = -jnp.log(x + z)


_fin = pl.pallas_call(
    _fin_body,
    grid=(_BATCH // _BB,),
    in_specs=[
        pl.BlockSpec((_BB, _SAMPLES), lambda i: (i, 0)),
        pl.BlockSpec((_BB, _SAMPLES - 1), lambda i: (i, 0)),
    ],
    out_specs=pl.BlockSpec((_BB, _SAMPLES - 1), lambda i: (i, 0)),
    out_shape=jax.ShapeDtypeStruct((_BATCH, _SAMPLES - 1), jnp.float32),
)


def kernel(inputs, weight):
    n2, dt = _sc_gather_reduce()(weight, inputs)
    return _fin(n2, dt)


# speedup vs baseline: 1.7903x; 1.7903x over previous
import functools

import jax
import jax.numpy as jnp
from jax import lax
from jax.experimental import pallas as pl
from jax.experimental.pallas import tpu as pltpu
from jax.experimental.pallas import tpu_sc as plsc

_EPS = 1e-5
_BOUNDARY = 1.0 - _EPS
_VOCAB = 1000000
_DIM = 32
_BATCH = 16384
_SAMPLES = 50

_NC, _NS = 2, 16
_NW = _NC * _NS
_BPW = _BATCH // _NW
_CB = 16
_NCH = _BPW // _CB
_HL = _DIM // 2


def _sc_body(table_hbm, idx_hbm, n2_hbm, dot_hbm,
             idx_v0, idx_v1, rows_v0, rows_v1, n2_v, dot_v, sem):
    wid = lax.axis_index("s") * _NC + lax.axis_index("c")
    base = wid * _BPW
    lanes = lax.iota(jnp.int32, 16)
    zero16 = jnp.zeros((16,), jnp.float32)
    idx_bufs = (idx_v0, idx_v1)
    row_bufs = (rows_v0, rows_v1)

    def fire(cc, idx_v, rows_v):
        b0 = base + cc * _CB
        pltpu.sync_copy(idx_hbm.at[pl.ds(b0, _CB)], idx_v)
        for b in range(_CB):
            pltpu.async_copy(
                table_hbm.at[idx_v.at[b]],
                rows_v.at[pl.ds(b * _SAMPLES, _SAMPLES)], sem)

    def compute(cc, rows_v):
        b0 = base + cc * _CB

        @plsc.parallel_loop(0, _CB, 1)
        def _bloop(b, rows_v=rows_v):
            rb = b * _SAMPLES
            bvec = lanes * 0 + b
            u_lo = rows_v[rb, pl.ds(0, _HL)]
            u_hi = rows_v[rb, pl.ds(_HL, _HL)]
            for g in range(4):
                n = 16 if g < 3 else _SAMPLES - 48

                @plsc.parallel_loop(0, n, 1, unroll=4,
                                    carry=(zero16, zero16))
                def acc(k, acc, g=g, u_lo=u_lo, u_hi=u_hi):
                    r = rb + 16 * g + k
                    lo = rows_v[r, pl.ds(0, _HL)]
                    hi = rows_v[r, pl.ds(_HL, _HL)]
                    n2s = jnp.sum(lo * lo + hi * hi)
                    dts = jnp.sum(u_lo * lo + u_hi * hi)
                    m = lanes == k
                    return (jnp.where(m, n2s, acc[0]),
                            jnp.where(m, dts, acc[1]))

                svec = lanes + 16 * g
                valid = lanes < n
                plsc.store_scatter(n2_v, [bvec, svec], acc[0], mask=valid)
                plsc.store_scatter(dot_v, [bvec, svec - 1], acc[1],
                                   mask=valid & (svec >= 1))
        pltpu.sync_copy(n2_v, n2_hbm.at[pl.ds(b0, _CB)])
        pltpu.sync_copy(dot_v, dot_hbm.at[pl.ds(b0, _CB)])

    def drain(rows_v):
        pltpu.make_async_copy(table_hbm.at[pl.ds(0, _CB * _SAMPLES)],
                              rows_v, sem).wait()

    fire(0, idx_bufs[0], row_bufs[0])

    def cbody(co, carry):
        for h in (0, 1):
            cc = 2 * co + h

            @pl.when(cc + 1 < _NCH)
            def _fire_next(cc=cc, h=h):
                fire(cc + 1, idx_bufs[h ^ 1], row_bufs[h ^ 1])

            drain(row_bufs[h])
            compute(cc, row_bufs[h])
        return carry

    lax.fori_loop(0, _NCH // 2, cbody, 0)


@functools.cache
def _sc_gather_reduce():
    return pl.kernel(
        _sc_body,
        out_type=(
            jax.ShapeDtypeStruct((_BATCH, _SAMPLES), jnp.float32),
            jax.ShapeDtypeStruct((_BATCH, _SAMPLES - 1), jnp.float32),
        ),
        mesh=plsc.VectorSubcoreMesh(
            core_axis_name="c", subcore_axis_name="s",
            num_cores=_NC, num_subcores=_NS,
        ),
        scratch_types=[
            pltpu.VMEM((_CB, _SAMPLES), jnp.int32),
            pltpu.VMEM((_CB, _SAMPLES), jnp.int32),
            pltpu.VMEM((_CB * _SAMPLES, _DIM), jnp.float32),
            pltpu.VMEM((_CB * _SAMPLES, _DIM), jnp.float32),
            pltpu.VMEM((_CB, _SAMPLES), jnp.float32),
            pltpu.VMEM((_CB, _SAMPLES - 1), jnp.float32),
            pltpu.SemaphoreType.DMA,
        ],
        compiler_params=pltpu.CompilerParams(
            use_tc_tiling_on_sc=False, needs_layout_passes=False),
    )


_BB = 2048


def _fin_body(n2_ref, dot_ref, out_ref):
    n2 = n2_ref[...]
    dt = dot_ref[...]
    n = jnp.sqrt(n2)
    scale = jnp.where(n > 1.0, 1.0 / (n + 1e-7), 1.0)
    sq_raw = n2 * scale * scale
    sq = jnp.clip(sq_raw, 0.0, _BOUNDARY)
    u2 = sq_raw[:, :1]
    v2 = sq_raw[:, 1:]
    su = scale[:, :1]
    sv = scale[:, 1:]
    sqdist = u2 + v2 - 2.0 * (su * sv) * dt
    squ = sq[:, :1]
    sqv = sq[:, 1:]
    x = sqdist / ((1.0 - squ) * (1.0 - sqv)) * 2.0 + 1.0
    z = jnp.sqrt(jnp.maximum(x * x - 1.0, 1e-12))
    out_ref[...] = -jnp.log(x + z)


_fin = pl.pallas_call(
    _fin_body,
    grid=(_BATCH // _BB,),
    in_specs=[
        pl.BlockSpec((_BB, _SAMPLES), lambda i: (i, 0)),
        pl.BlockSpec((_BB, _SAMPLES - 1), lambda i: (i, 0)),
    ],
    out_specs=pl.BlockSpec((_BB, _SAMPLES - 1), lambda i: (i, 0)),
    out_shape=jax.ShapeDtypeStruct((_BATCH, _SAMPLES - 1), jnp.float32),
)


def kernel(inputs, weight):
    n2, dt = _sc_gather_reduce()(weight, inputs)
    return _fin(n2, dt)


# speedup vs baseline: 1.9078x; 1.0656x over previous
import functools

import jax
import jax.numpy as jnp
from jax import lax
from jax.experimental import pallas as pl
from jax.experimental.pallas import tpu as pltpu
from jax.experimental.pallas import tpu_sc as plsc

_EPS = 1e-5
_BOUNDARY = 1.0 - _EPS
_VOCAB = 1000000
_DIM = 32
_BATCH = 16384
_SAMPLES = 50

_NC, _NS = 2, 16
_NW = _NC * _NS
_BPW = _BATCH // _NW
_CB = 32
_NCH = _BPW // _CB
_HL = _DIM // 2


def _sc_body(table_hbm, idx_hbm, n2_hbm, dot_hbm,
             idx_v0, idx_v1, rows_v0, rows_v1, n2_v, dot_v, sem):
    wid = lax.axis_index("s") * _NC + lax.axis_index("c")
    base = wid * _BPW
    lanes = lax.iota(jnp.int32, 16)
    zero16 = jnp.zeros((16,), jnp.float32)
    idx_bufs = (idx_v0, idx_v1)
    row_bufs = (rows_v0, rows_v1)

    def fire(cc, idx_v, rows_v):
        b0 = base + cc * _CB
        pltpu.sync_copy(idx_hbm.at[pl.ds(b0, _CB)], idx_v)
        for b in range(_CB):
            pltpu.async_copy(
                table_hbm.at[idx_v.at[b]],
                rows_v.at[pl.ds(b * _SAMPLES, _SAMPLES)], sem)

    def compute(cc, rows_v):
        b0 = base + cc * _CB

        @plsc.parallel_loop(0, _CB, 1, unroll=4)
        def _bloop(b, rows_v=rows_v):
            rb = b * _SAMPLES
            bvec = lanes * 0 + b
            u_lo = rows_v[rb, pl.ds(0, _HL)]
            u_hi = rows_v[rb, pl.ds(_HL, _HL)]
            for g in range(4):
                n = 16 if g < 3 else _SAMPLES - 48

                @plsc.parallel_loop(0, n, 1, unroll=16,
                                    carry=(zero16, zero16))
                def acc(k, acc, g=g, u_lo=u_lo, u_hi=u_hi):
                    r = rb + 16 * g + k
                    lo = rows_v[r, pl.ds(0, _HL)]
                    hi = rows_v[r, pl.ds(_HL, _HL)]
                    n2s = jnp.sum(lo * lo + hi * hi)
                    dts = jnp.sum(u_lo * lo + u_hi * hi)
                    m = lanes == k
                    return (jnp.where(m, n2s, acc[0]),
                            jnp.where(m, dts, acc[1]))

                svec = lanes + 16 * g
                valid = lanes < n
                plsc.store_scatter(n2_v, [bvec, svec], acc[0], mask=valid)
                plsc.store_scatter(dot_v, [bvec, svec - 1], acc[1],
                                   mask=valid & (svec >= 1))
        pltpu.sync_copy(n2_v, n2_hbm.at[pl.ds(b0, _CB)])
        pltpu.sync_copy(dot_v, dot_hbm.at[pl.ds(b0, _CB)])

    def drain(rows_v):
        pltpu.make_async_copy(table_hbm.at[pl.ds(0, _CB * _SAMPLES)],
                              rows_v, sem).wait()

    fire(0, idx_bufs[0], row_bufs[0])

    def cbody(co, carry):
        for h in (0, 1):
            cc = 2 * co + h

            @pl.when(cc + 1 < _NCH)
            def _fire_next(cc=cc, h=h):
                fire(cc + 1, idx_bufs[h ^ 1], row_bufs[h ^ 1])

            drain(row_bufs[h])
            compute(cc, row_bufs[h])
        return carry

    lax.fori_loop(0, _NCH // 2, cbody, 0)


@functools.cache
def _sc_gather_reduce():
    return pl.kernel(
        _sc_body,
        out_type=(
            jax.ShapeDtypeStruct((_BATCH, _SAMPLES), jnp.float32),
            jax.ShapeDtypeStruct((_BATCH, _SAMPLES - 1), jnp.float32),
        ),
        mesh=plsc.VectorSubcoreMesh(
            core_axis_name="c", subcore_axis_name="s",
            num_cores=_NC, num_subcores=_NS,
        ),
        scratch_types=[
            pltpu.VMEM((_CB, _SAMPLES), jnp.int32),
            pltpu.VMEM((_CB, _SAMPLES), jnp.int32),
            pltpu.VMEM((_CB * _SAMPLES, _DIM), jnp.float32),
            pltpu.VMEM((_CB * _SAMPLES, _DIM), jnp.float32),
            pltpu.VMEM((_CB, _SAMPLES), jnp.float32),
            pltpu.VMEM((_CB, _SAMPLES - 1), jnp.float32),
            pltpu.SemaphoreType.DMA,
        ],
        compiler_params=pltpu.CompilerParams(
            use_tc_tiling_on_sc=False, needs_layout_passes=False),
    )


_BB = 4096


def _fin_body(n2_ref, dot_ref, out_ref):
    n2 = n2_ref[...]
    dt = dot_ref[...]
    n = jnp.sqrt(n2)
    scale = jnp.where(n > 1.0, 1.0 / (n + 1e-7), 1.0)
    sq_raw = n2 * scale * scale
    sq = jnp.clip(sq_raw, 0.0, _BOUNDARY)
    u2 = sq_raw[:, :1]
    v2 = sq_raw[:, 1:]
    su = scale[:, :1]
    sv = scale[:, 1:]
    sqdist = u2 + v2 - 2.0 * (su * sv) * dt
    squ = sq[:, :1]
    sqv = sq[:, 1:]
    x = sqdist / ((1.0 - squ) * (1.0 - sqv)) * 2.0 + 1.0
    z = jnp.sqrt(jnp.maximum(x * x - 1.0, 1e-12))
    out_ref[...] = -jnp.log(x + z)


_fin = pl.pallas_call(
    _fin_body,
    grid=(_BATCH // _BB,),
    in_specs=[
        pl.BlockSpec((_BB, _SAMPLES), lambda i: (i, 0)),
        pl.BlockSpec((_BB, _SAMPLES - 1), lambda i: (i, 0)),
    ],
    out_specs=pl.BlockSpec((_BB, _SAMPLES - 1), lambda i: (i, 0)),
    out_shape=jax.ShapeDtypeStruct((_BATCH, _SAMPLES - 1), jnp.float32),
)


def kernel(inputs, weight):
    n2, dt = _sc_gather_reduce()(weight, inputs)
    return _fin(n2, dt)


# speedup vs baseline: 1.9239x; 1.0084x over previous
import functools

import jax
import jax.numpy as jnp
from jax import lax
from jax.experimental import pallas as pl
from jax.experimental.pallas import tpu as pltpu
from jax.experimental.pallas import tpu_sc as plsc

_EPS = 1e-5
_BOUNDARY = 1.0 - _EPS
_VOCAB = 1000000
_DIM = 32
_BATCH = 16384
_SAMPLES = 50

_NC, _NS = 2, 16
_NW = _NC * _NS
_BPW = _BATCH // _NW
_CB = 32
_NCH = _BPW // _CB
_HL = _DIM // 2


def _sc_body(table_hbm, idx_hbm, n2_hbm, dot_hbm,
             idx_v0, idx_v1, rows_v0, rows_v1, n2_v, dot_v, sem):
    wid = lax.axis_index("s") * _NC + lax.axis_index("c")
    base = wid * _BPW
    lanes = lax.iota(jnp.int32, 16)
    zero16 = jnp.zeros((16,), jnp.float32)
    idx_bufs = (idx_v0, idx_v1)
    row_bufs = (rows_v0, rows_v1)

    def fire(cc, idx_v, rows_v):
        b0 = base + cc * _CB
        pltpu.sync_copy(idx_hbm.at[pl.ds(b0, _CB)], idx_v)
        for b in range(_CB):
            pltpu.async_copy(
                table_hbm.at[idx_v.at[b]],
                rows_v.at[pl.ds(b * _SAMPLES, _SAMPLES)], sem)

    def compute(cc, rows_v):
        b0 = base + cc * _CB

        @plsc.parallel_loop(0, _CB, 1, unroll=2)
        def _bloop(b, rows_v=rows_v):
            rb = b * _SAMPLES
            bvec = lanes * 0 + b
            u_lo = rows_v[rb, pl.ds(0, _HL)]
            u_hi = rows_v[rb, pl.ds(_HL, _HL)]
            for g in range(4):
                n = 16 if g < 3 else _SAMPLES - 48

                @plsc.parallel_loop(0, n, 1, unroll=16,
                                    carry=(zero16, zero16))
                def acc(k, acc, g=g, u_lo=u_lo, u_hi=u_hi):
                    r = rb + 16 * g + k
                    lo = rows_v[r, pl.ds(0, _HL)]
                    hi = rows_v[r, pl.ds(_HL, _HL)]
                    n2s = jnp.sum(lo * lo + hi * hi)
                    dts = jnp.sum(u_lo * lo + u_hi * hi)
                    m = lanes == k
                    return (jnp.where(m, n2s, acc[0]),
                            jnp.where(m, dts, acc[1]))

                svec = lanes + 16 * g
                valid = lanes < n
                plsc.store_scatter(n2_v, [bvec, svec], acc[0], mask=valid)
                plsc.store_scatter(dot_v, [bvec, svec - 1], acc[1],
                                   mask=valid & (svec >= 1))
        pltpu.sync_copy(n2_v, n2_hbm.at[pl.ds(b0, _CB)])
        pltpu.sync_copy(dot_v, dot_hbm.at[pl.ds(b0, _CB)])

    def drain(rows_v):
        pltpu.make_async_copy(table_hbm.at[pl.ds(0, _CB * _SAMPLES)],
                              rows_v, sem).wait()

    fire(0, idx_bufs[0], row_bufs[0])

    def cbody(co, carry):
        for h in (0, 1):
            cc = 2 * co + h

            @pl.when(cc + 1 < _NCH)
            def _fire_next(cc=cc, h=h):
                fire(cc + 1, idx_bufs[h ^ 1], row_bufs[h ^ 1])

            drain(row_bufs[h])
            compute(cc, row_bufs[h])
        return carry

    lax.fori_loop(0, _NCH // 2, cbody, 0)


@functools.cache
def _sc_gather_reduce():
    return pl.kernel(
        _sc_body,
        out_type=(
            jax.ShapeDtypeStruct((_BATCH, _SAMPLES), jnp.float32),
            jax.ShapeDtypeStruct((_BATCH, _SAMPLES - 1), jnp.float32),
        ),
        mesh=plsc.VectorSubcoreMesh(
            core_axis_name="c", subcore_axis_name="s",
            num_cores=_NC, num_subcores=_NS,
        ),
        scratch_types=[
            pltpu.VMEM((_CB, _SAMPLES), jnp.int32),
            pltpu.VMEM((_CB, _SAMPLES), jnp.int32),
            pltpu.VMEM((_CB * _SAMPLES, _DIM), jnp.float32),
            pltpu.VMEM((_CB * _SAMPLES, _DIM), jnp.float32),
            pltpu.VMEM((_CB, _SAMPLES), jnp.float32),
            pltpu.VMEM((_CB, _SAMPLES - 1), jnp.float32),
            pltpu.SemaphoreType.DMA,
        ],
        compiler_params=pltpu.CompilerParams(
            use_tc_tiling_on_sc=False, needs_layout_passes=False),
    )


_BB = 4096


def _fin_body(n2_ref, dot_ref, out_ref):
    n2 = n2_ref[...]
    dt = dot_ref[...]
    n = jnp.sqrt(n2)
    scale = jnp.where(n > 1.0, 1.0 / (n + 1e-7), 1.0)
    sq_raw = n2 * scale * scale
    sq = jnp.clip(sq_raw, 0.0, _BOUNDARY)
    u2 = sq_raw[:, :1]
    v2 = sq_raw[:, 1:]
    su = scale[:, :1]
    sv = scale[:, 1:]
    sqdist = u2 + v2 - 2.0 * (su * sv) * dt
    squ = sq[:, :1]
    sqv = sq[:, 1:]
    x = sqdist / ((1.0 - squ) * (1.0 - sqv)) * 2.0 + 1.0
    z = jnp.sqrt(jnp.maximum(x * x - 1.0, 1e-12))
    out_ref[...] = -jnp.log(x + z)


_fin = pl.pallas_call(
    _fin_body,
    grid=(_BATCH // _BB,),
    in_specs=[
        pl.BlockSpec((_BB, _SAMPLES), lambda i: (i, 0)),
        pl.BlockSpec((_BB, _SAMPLES - 1), lambda i: (i, 0)),
    ],
    out_specs=pl.BlockSpec((_BB, _SAMPLES - 1), lambda i: (i, 0)),
    out_shape=jax.ShapeDtypeStruct((_BATCH, _SAMPLES - 1), jnp.float32),
)


def kernel(inputs, weight):
    n2, dt = _sc_gather_reduce()(weight, inputs)
    return _fin(n2, dt)
